# baseline (speedup 1.0000x reference)
import functools

import jax
import jax.numpy as jnp
from jax import lax
from jax.experimental import pallas as pl
from jax.experimental.pallas import tpu as pltpu
from jax.experimental.pallas import tpu_sc as plsc

_BATCH = 16384
_D = 32
_RANK = 16
_NC = 2
_NS = 16
_NW = _NC * _NS
_BPW = _BATCH // _NW


def _gather_body(users_hbm, items_hbm, wu_hbm, wi_hbm, a_hbm, ub_hbm, ib_hbm,
                 u_out, wi_out, a_out, ub_out, ib_out,
                 uidx_v, iidx_v, u_v, wi_v, a_v, ub_v, ib_v,
                 s0, s1, s2, s3, s4):
    wid = lax.axis_index("s") * _NC + lax.axis_index("c")
    base = wid * _BPW
    pltpu.sync_copy(users_hbm.at[pl.ds(base, _BPW)], uidx_v)
    pltpu.sync_copy(items_hbm.at[pl.ds(base, _BPW)], iidx_v)
    cps = [
        pltpu.async_copy(wu_hbm.at[uidx_v], u_v, s0),
        pltpu.async_copy(wi_hbm.at[iidx_v], wi_v, s1),
        pltpu.async_copy(a_hbm.at[iidx_v], a_v, s2),
        pltpu.async_copy(ub_hbm.at[uidx_v], ub_v, s3),
        pltpu.async_copy(ib_hbm.at[iidx_v], ib_v, s4),
    ]
    for c in cps:
        c.wait()
    pltpu.sync_copy(u_v, u_out.at[pl.ds(base, _BPW)])
    pltpu.sync_copy(wi_v, wi_out.at[pl.ds(base, _BPW)])
    pltpu.sync_copy(a_v, a_out.at[pl.ds(base, _BPW)])
    pltpu.sync_copy(ub_v, ub_out.at[pl.ds(base, _BPW)])
    pltpu.sync_copy(ib_v, ib_out.at[pl.ds(base, _BPW)])


_sc_gather = functools.partial(
    pl.kernel,
    out_type=(
        jax.ShapeDtypeStruct((_BATCH, _D), jnp.bfloat16),
        jax.ShapeDtypeStruct((_BATCH, _D), jnp.float32),
        jax.ShapeDtypeStruct((_BATCH, _RANK), jnp.float32),
        jax.ShapeDtypeStruct((_BATCH,), jnp.float32),
        jax.ShapeDtypeStruct((_BATCH,), jnp.float32),
    ),
    mesh=plsc.VectorSubcoreMesh(core_axis_name="c", subcore_axis_name="s"),
    compiler_params=pltpu.CompilerParams(use_tc_tiling_on_sc=False),
    scratch_types=[
        pltpu.VMEM((_BPW,), jnp.int32),
        pltpu.VMEM((_BPW,), jnp.int32),
        pltpu.VMEM((_BPW, _D), jnp.bfloat16),
        pltpu.VMEM((_BPW, _D), jnp.float32),
        pltpu.VMEM((_BPW, _RANK), jnp.float32),
        pltpu.VMEM((_BPW,), jnp.float32),
        pltpu.VMEM((_BPW,), jnp.float32),
        pltpu.SemaphoreType.DMA,
        pltpu.SemaphoreType.DMA,
        pltpu.SemaphoreType.DMA,
        pltpu.SemaphoreType.DMA,
        pltpu.SemaphoreType.DMA,
    ],
)(_gather_body)


def _finish_body(u_ref, wi_ref, a_ref, ub_ref, ib_ref, b_ref, out_ref):
    corr = jnp.dot(a_ref[...], b_ref[...], preferred_element_type=jnp.float32)
    out_ref[...] = (jnp.sum(u_ref[...].astype(jnp.float32)
                            * (wi_ref[...] + corr), axis=1)
                    + ub_ref[...] + ib_ref[...])


def kernel(users, items, W_user, W_item, user_bias, item_bias, A, B):
    users = users.astype(jnp.int32)
    items = items.astype(jnp.int32)
    u, wi, a, ub, ib = _sc_gather(users, items,
                                  W_user.astype(jnp.bfloat16), W_item, A,
                                  user_bias, item_bias)
    pred = pl.pallas_call(
        _finish_body,
        out_shape=jax.ShapeDtypeStruct((_BATCH,), jnp.float32),
    )(u, wi, a, ub, ib, B)
    return pred


# speedup vs baseline: 1.1771x; 1.1771x over previous
import functools

import jax
import jax.numpy as jnp
from jax import lax
from jax.experimental import pallas as pl
from jax.experimental.pallas import tpu as pltpu
from jax.experimental.pallas import tpu_sc as plsc

_BATCH = 16384
_D = 32
_RANK = 16
_NC = 2
_NS = 16
_NW = _NC * _NS
_BPW = _BATCH // _NW


def _gather_body(users_hbm, items_hbm, wu_hbm, wi_hbm, a_hbm, ub_hbm, ib_hbm,
                 u_out, wi_out, a_out, ub_out, ib_out,
                 uidx_v, iidx_v, u_v, wi_v, a_v, ub_v, ib_v,
                 s0, s1, s2, s3, s4):
    wid = lax.axis_index("s") * _NC + lax.axis_index("c")
    base = wid * _BPW
    pltpu.sync_copy(users_hbm.at[pl.ds(base, _BPW)], uidx_v)
    pltpu.sync_copy(items_hbm.at[pl.ds(base, _BPW)], iidx_v)
    cps = [
        pltpu.async_copy(wu_hbm.at[uidx_v], u_v, s0),
        pltpu.async_copy(wi_hbm.at[iidx_v], wi_v, s1),
        pltpu.async_copy(a_hbm.at[iidx_v], a_v, s2),
        pltpu.async_copy(ub_hbm.at[uidx_v], ub_v, s3),
        pltpu.async_copy(ib_hbm.at[iidx_v], ib_v, s4),
    ]
    for c in cps:
        c.wait()
    pltpu.sync_copy(u_v, u_out.at[pl.ds(base, _BPW)])
    pltpu.sync_copy(wi_v, wi_out.at[pl.ds(base, _BPW)])
    pltpu.sync_copy(a_v, a_out.at[pl.ds(base, _BPW)])
    pltpu.sync_copy(ub_v, ub_out.at[pl.ds(base, _BPW)])
    pltpu.sync_copy(ib_v, ib_out.at[pl.ds(base, _BPW)])


_sc_gather = functools.partial(
    pl.kernel,
    out_type=(
        jax.ShapeDtypeStruct((_BATCH, _D), jnp.float32),
        jax.ShapeDtypeStruct((_BATCH, _D), jnp.float32),
        jax.ShapeDtypeStruct((_BATCH, _RANK), jnp.float32),
        jax.ShapeDtypeStruct((_BATCH,), jnp.float32),
        jax.ShapeDtypeStruct((_BATCH,), jnp.float32),
    ),
    mesh=plsc.VectorSubcoreMesh(core_axis_name="c", subcore_axis_name="s"),
    compiler_params=pltpu.CompilerParams(use_tc_tiling_on_sc=False),
    scratch_types=[
        pltpu.VMEM((_BPW,), jnp.int32),
        pltpu.VMEM((_BPW,), jnp.int32),
        pltpu.VMEM((_BPW, _D), jnp.float32),
        pltpu.VMEM((_BPW, _D), jnp.float32),
        pltpu.VMEM((_BPW, _RANK), jnp.float32),
        pltpu.VMEM((_BPW,), jnp.float32),
        pltpu.VMEM((_BPW,), jnp.float32),
        pltpu.SemaphoreType.DMA,
        pltpu.SemaphoreType.DMA,
        pltpu.SemaphoreType.DMA,
        pltpu.SemaphoreType.DMA,
        pltpu.SemaphoreType.DMA,
    ],
)(_gather_body)


def _finish_body(u_ref, wi_ref, a_ref, ub_ref, ib_ref, b_ref, out_ref):
    corr = jnp.dot(a_ref[...], b_ref[...], preferred_element_type=jnp.float32)
    out_ref[...] = (jnp.sum(u_ref[...] * (wi_ref[...] + corr), axis=1)
                    + ub_ref[...] + ib_ref[...])


def kernel(users, items, W_user, W_item, user_bias, item_bias, A, B):
    users = users.astype(jnp.int32)
    items = items.astype(jnp.int32)
    u, wi, a, ub, ib = _sc_gather(users, items, W_user, W_item, A,
                                  user_bias, item_bias)
    pred = pl.pallas_call(
        _finish_body,
        out_shape=jax.ShapeDtypeStruct((_BATCH,), jnp.float32),
    )(u, wi, a, ub, ib, B)
    return pred
